# baseline (speedup 1.0000x reference)
import jax
import jax.numpy as jnp
from jax.experimental import pallas as pl
from jax.experimental.pallas import tpu as pltpu

_CHUNK_B = 2
_RING = 6
_LOOKAHEAD = 3


def _body(pos_ref, kv_ref, cache_ref, out_ref, buf, in_sems, out_sems, tok_sem):
    n_chunks = cache_ref.shape[0] // _CHUNK_B
    in_cps = [None] * n_chunks
    out_cps = [None] * n_chunks
    started = [0]

    def maybe_start_read():
        c = started[0]
        if c >= n_chunks:
            return
        s = c % _RING
        if c >= _RING:
            out_cps[c - _RING].wait()
        cp = pltpu.make_async_copy(
            cache_ref.at[pl.ds(c * _CHUNK_B, _CHUNK_B)], buf.at[s], in_sems.at[s]
        )
        cp.start()
        in_cps[c] = cp
        started[0] = c + 1

    for _ in range(_LOOKAHEAD):
        maybe_start_read()
    for c in range(n_chunks):
        s = c % _RING
        in_cps[c].wait()
        cp = pltpu.make_async_copy(
            buf.at[s], out_ref.at[pl.ds(c * _CHUNK_B, _CHUNK_B)], out_sems.at[s]
        )
        cp.start()
        out_cps[c] = cp
        maybe_start_read()
    for c in range(n_chunks - _RING, n_chunks):
        if out_cps[c] is not None and c >= 0:
            out_cps[c].wait()

    tok = pltpu.make_async_copy(kv_ref, out_ref.at[:, pl.ds(pos_ref[0], 1), :], tok_sem)
    tok.start()
    tok.wait()


def kernel(kv, start_pos, kv_cache):
    bsz, _, head = kv.shape
    win = kv_cache.shape[1]
    pos = jnp.reshape(jnp.asarray(start_pos, jnp.int32) % win, (1,))
    cache = kv_cache[:bsz]
    out = pl.pallas_call(
        _body,
        out_shape=jax.ShapeDtypeStruct(cache.shape, cache.dtype),
        in_specs=[
            pl.BlockSpec(memory_space=pltpu.SMEM),
            pl.BlockSpec(memory_space=pltpu.VMEM),
            pl.BlockSpec(memory_space=pltpu.HBM),
        ],
        out_specs=pl.BlockSpec(memory_space=pltpu.HBM),
        scratch_shapes=[
            pltpu.VMEM((_RING, _CHUNK_B, win, head), jnp.float32),
            pltpu.SemaphoreType.DMA((_RING,)),
            pltpu.SemaphoreType.DMA((_RING,)),
            pltpu.SemaphoreType.DMA,
        ],
    )(pos, kv, cache)
    return out


# speedup vs baseline: 1.0061x; 1.0061x over previous
import jax
import jax.numpy as jnp
from jax.experimental import pallas as pl
from jax.experimental.pallas import tpu as pltpu

_B_BLK = 8
_W_BLK = 2048


def _body(pos_ref, kv_ref, cache_ref, out_ref):
    j = pl.program_id(1)
    local = pos_ref[0] - j * _W_BLK
    out_ref[...] = cache_ref[...]

    @pl.when((local >= 0) & (local < _W_BLK))
    def _():
        out_ref[:, pl.ds(local, 1), :] = kv_ref[...]


def kernel(kv, start_pos, kv_cache):
    bsz, _, head = kv.shape
    win = kv_cache.shape[1]
    pos = jnp.reshape(jnp.asarray(start_pos, jnp.int32) % win, (1,))
    cache = kv_cache[:bsz]
    out = pl.pallas_call(
        _body,
        grid=(bsz // _B_BLK, win // _W_BLK),
        out_shape=jax.ShapeDtypeStruct(cache.shape, cache.dtype),
        in_specs=[
            pl.BlockSpec(memory_space=pltpu.SMEM),
            pl.BlockSpec((_B_BLK, 1, head), lambda i, j: (i, 0, 0)),
            pl.BlockSpec((_B_BLK, _W_BLK, head), lambda i, j: (i, j, 0)),
        ],
        out_specs=pl.BlockSpec((_B_BLK, _W_BLK, head), lambda i, j: (i, j, 0)),
    )(pos, kv, cache)
    return out
